# Initial kernel scaffold; baseline (speedup 1.0000x reference)
import jax
import jax.numpy as jnp
from jax.experimental import pallas as pl


def kernel(x, edge_index, Wl1, bl1, Wr1, br1, att1, bias1, Wl2, bl2, Wr2, br2, att2, bias2, Wm1, bm1, Wm2, bm2):
    raise NotImplementedError("write your pallas kernel here")


# speedup vs baseline: 1.5566x; 1.5566x over previous
import functools

import jax
import jax.numpy as jnp
from jax.experimental import pallas as pl
from jax.experimental.pallas import tpu as pltpu

N = 10000
D = 128


def _matmul_kernel(x_ref, w_ref, b_ref, o_ref):
    o_ref[...] = (
        jnp.dot(x_ref[...], w_ref[...], preferred_element_type=jnp.float32)
        + b_ref[...]
    )


def _dense(x, w, b):
    n, k = x.shape
    m = w.shape[1]
    return pl.pallas_call(
        _matmul_kernel,
        out_shape=jax.ShapeDtypeStruct((n, m), jnp.float32),
    )(x, w, b.reshape(1, m))


def _head_kernel(h_ref, wm1_ref, bm1_ref, wm2_ref, bm2_ref, o_ref):
    h = h_ref[...]
    mean_pool = jnp.mean(h, axis=0, keepdims=True)
    max_pool = jnp.max(h, axis=0, keepdims=True)
    emb = jnp.concatenate([mean_pool, max_pool], axis=-1)
    s = jax.nn.relu(
        jnp.dot(emb, wm1_ref[...], preferred_element_type=jnp.float32)
        + bm1_ref[...]
    )
    o_ref[...] = jnp.dot(s, wm2_ref[...], preferred_element_type=jnp.float32) + bm2_ref[...]


def _head(h, Wm1, bm1, Wm2, bm2):
    out = pl.pallas_call(
        _head_kernel,
        out_shape=jax.ShapeDtypeStruct((1, 1), jnp.float32),
    )(h, Wm1, bm1.reshape(1, -1), Wm2, bm2.reshape(1, 1))
    return out[0]


def _gatv2_edges(xl, xr, src, dst, att):
    n = xl.shape[0]
    e = jax.nn.leaky_relu(xl[src] + xr[dst], negative_slope=0.2)
    logit = jnp.sum(e * att, axis=-1)
    m = jnp.max(logit)
    unnorm = jnp.exp(logit - m)
    denom = jax.ops.segment_sum(unnorm, dst, num_segments=n)
    alpha = unnorm / (denom[dst] + 1e-16)
    return jax.ops.segment_sum(alpha[:, None] * xl[src], dst, num_segments=n)


def kernel(x, edge_index, Wl1, bl1, Wr1, br1, att1, bias1,
           Wl2, bl2, Wr2, br2, att2, bias2, Wm1, bm1, Wm2, bm2):
    n = x.shape[0]
    loop = jnp.arange(n, dtype=edge_index.dtype)
    src = jnp.concatenate([edge_index[0], loop])
    dst = jnp.concatenate([edge_index[1], loop])

    xl1 = _dense(x, Wl1, bl1)
    xr1 = _dense(x, Wr1, br1)
    h = _gatv2_edges(xl1, xr1, src, dst, att1) + bias1
    h = jax.nn.relu(h)

    xl2 = _dense(h, Wl2, bl2)
    xr2 = _dense(h, Wr2, br2)
    h2 = _gatv2_edges(xl2, xr2, src, dst, att2) + bias2

    return _head(h2, Wm1, bm1, Wm2, bm2)


# speedup vs baseline: 1.7332x; 1.1135x over previous
import functools

import jax
import jax.numpy as jnp
from jax import lax
from jax.experimental import pallas as pl
from jax.experimental.pallas import tpu as pltpu
from jax.experimental.pallas import tpu_sc as plsc

N = 10000
D = 128
E_RAW = 320000
ETOT = E_RAW + N
NC, NS, L = 2, 16, 16
NW = NC * NS
W_E = 192
CHUNK = 10368
EP = CHUNK * NW
N_WIN = CHUNK // W_E
G = W_E // L


def _matmul_kernel(x_ref, w_ref, b_ref, o_ref):
    o_ref[...] = (
        jnp.dot(x_ref[...], w_ref[...], preferred_element_type=jnp.float32)
        + b_ref[...]
    )


def _dense(x, w, b):
    n, k = x.shape
    m = w.shape[1]
    return pl.pallas_call(
        _matmul_kernel,
        out_shape=jax.ShapeDtypeStruct((n, m), jnp.float32),
    )(x, w, b.reshape(1, m))


def _head_kernel(h_ref, wm1_ref, bm1_ref, wm2_ref, bm2_ref, o_ref):
    h = h_ref[...]
    mean_pool = jnp.mean(h, axis=0, keepdims=True)
    max_pool = jnp.max(h, axis=0, keepdims=True)
    emb = jnp.concatenate([mean_pool, max_pool], axis=-1)
    s = jax.nn.relu(
        jnp.dot(emb, wm1_ref[...], preferred_element_type=jnp.float32)
        + bm1_ref[...]
    )
    o_ref[...] = jnp.dot(s, wm2_ref[...], preferred_element_type=jnp.float32) + bm2_ref[...]


def _head(h, Wm1, bm1, Wm2, bm2):
    out = pl.pallas_call(
        _head_kernel,
        out_shape=jax.ShapeDtypeStruct((1, 1), jnp.float32),
    )(h, Wm1, bm1.reshape(1, -1), Wm2, bm2.reshape(1, 1))
    return out[0]


def _lane_shuffle(v, idx):
    dnums = lax.GatherDimensionNumbers(
        offset_dims=(), collapsed_slice_dims=(0,), start_index_map=(0,))
    return lax.gather(v, idx[:, None], dnums, slice_sizes=(1,),
                      mode=lax.GatherScatterMode.PROMISE_IN_BOUNDS)


def _sc_logits(xl, xr, src_p, dst_p, att):
    mesh = plsc.VectorSubcoreMesh(core_axis_name="c", subcore_axis_name="s")

    @functools.partial(
        pl.kernel,
        out_type=[
            jax.ShapeDtypeStruct((EP,), jnp.float32),
            jax.ShapeDtypeStruct((NW, L), jnp.float32),
        ],
        mesh=mesh,
        scratch_types=[
            pltpu.VMEM((W_E,), jnp.int32),
            pltpu.VMEM((W_E,), jnp.int32),
            pltpu.VMEM((W_E, D), jnp.float32),
            pltpu.VMEM((W_E, D), jnp.float32),
            pltpu.VMEM((W_E,), jnp.float32),
            pltpu.VMEM((L,), jnp.float32),
            pltpu.VMEM((D,), jnp.float32),
            pltpu.SemaphoreType.DMA,
            pltpu.SemaphoreType.DMA,
        ],
    )
    def k(xl_h, xr_h, src_h, dst_h, att_h, lg_h, mx_h,
          src_v, dst_v, xlr, xrr, lg_v, mx_v, att_v, s1, s2):
        cid = lax.axis_index("c")
        sid = lax.axis_index("s")
        wid = sid * NC + cid
        base = wid * CHUNK
        pltpu.sync_copy(att_h, att_v)
        iota = lax.iota(jnp.int32, L)
        att_c = [att_v[pl.ds(c * L, L)] for c in range(D // L)]

        def window(w, running_max):
            off = base + w * W_E
            pltpu.sync_copy(src_h.at[pl.ds(off, W_E)], src_v)
            pltpu.sync_copy(dst_h.at[pl.ds(off, W_E)], dst_v)
            cl = pltpu.async_copy(xl_h.at[src_v], xlr, s1)
            cr = pltpu.async_copy(xr_h.at[dst_v], xrr, s2)
            cl.wait()
            cr.wait()

            def gbody(g, running_max):
                e0 = g * L
                lvec = jnp.zeros((L,), jnp.float32)
                for i in range(L):
                    acc = jnp.zeros((L,), jnp.float32)
                    for c in range(D // L):
                        vl = xlr[e0 + i, pl.ds(c * L, L)]
                        vr = xrr[e0 + i, pl.ds(c * L, L)]
                        v = vl + vr
                        v = jnp.maximum(v, 0.2 * v)
                        acc = acc + v * att_c[c]
                    for sh in (8, 4, 2, 1):
                        acc = acc + _lane_shuffle(acc, iota ^ sh)
                    lvec = jnp.where(iota == i, acc, lvec)
                lg_v[pl.ds(e0, L)] = lvec
                return jnp.maximum(running_max, lvec)

            wmax = lax.fori_loop(0, G, gbody, running_max)
            pltpu.sync_copy(lg_v, lg_h.at[pl.ds(off, W_E)])
            return wmax

        neg = jnp.full((L,), -3e38, jnp.float32)
        running_max = lax.fori_loop(0, N_WIN, window, neg)
        mx_v[...] = running_max
        pltpu.sync_copy(mx_v, mx_h.at[wid])

    return k(xl, xr, src_p, dst_p, att)


def _gatv2_edges(xl, xr, src_p, dst_p, src, dst, att):
    n = xl.shape[0]
    logits_p, maxes = _sc_logits(xl, xr, src_p, dst_p, att)
    logit = logits_p[:ETOT]
    m = jnp.max(maxes)
    unnorm = jnp.exp(logit - m)
    denom = jax.ops.segment_sum(unnorm, dst, num_segments=n)
    alpha = unnorm / (denom[dst] + 1e-16)
    return jax.ops.segment_sum(alpha[:, None] * xl[src], dst, num_segments=n)


def kernel(x, edge_index, Wl1, bl1, Wr1, br1, att1, bias1,
           Wl2, bl2, Wr2, br2, att2, bias2, Wm1, bm1, Wm2, bm2):
    n = x.shape[0]
    loop = jnp.arange(n, dtype=edge_index.dtype)
    src = jnp.concatenate([edge_index[0], loop])
    dst = jnp.concatenate([edge_index[1], loop])
    pad = jnp.arange(EP - ETOT, dtype=src.dtype) % n
    src_p = jnp.concatenate([src, pad])
    dst_p = jnp.concatenate([dst, pad])

    xl1 = _dense(x, Wl1, bl1)
    xr1 = _dense(x, Wr1, br1)
    h = _gatv2_edges(xl1, xr1, src_p, dst_p, src, dst, att1) + bias1
    h = jax.nn.relu(h)

    xl2 = _dense(h, Wl2, bl2)
    xr2 = _dense(h, Wr2, br2)
    h2 = _gatv2_edges(xl2, xr2, src_p, dst_p, src, dst, att2) + bias2

    return _head(h2, Wm1, bm1, Wm2, bm2)


# speedup vs baseline: 11.2046x; 6.4647x over previous
import functools

import jax
import jax.numpy as jnp
from jax import lax
from jax.experimental import pallas as pl
from jax.experimental.pallas import tpu as pltpu
from jax.experimental.pallas import tpu_sc as plsc

N = 10000
D = 128
E_RAW = 320000
ETOT = E_RAW + N
NC, NS, L = 2, 16, 16
NW = NC * NS
W_E = 192
CHUNK = 10368
EP = CHUNK * NW
N_WIN = CHUNK // W_E
G = W_E // L


def _matmul_kernel(x_ref, w_ref, b_ref, o_ref):
    o_ref[...] = (
        jnp.dot(x_ref[...], w_ref[...], preferred_element_type=jnp.float32)
        + b_ref[...]
    )


def _dense(x, w, b):
    n, k = x.shape
    m = w.shape[1]
    return pl.pallas_call(
        _matmul_kernel,
        out_shape=jax.ShapeDtypeStruct((n, m), jnp.float32),
    )(x, w, b.reshape(1, m))


def _head_kernel(h_ref, wm1_ref, bm1_ref, wm2_ref, bm2_ref, o_ref):
    h = h_ref[...]
    mean_pool = jnp.mean(h, axis=0, keepdims=True)
    max_pool = jnp.max(h, axis=0, keepdims=True)
    emb = jnp.concatenate([mean_pool, max_pool], axis=-1)
    s = jax.nn.relu(
        jnp.dot(emb, wm1_ref[...], preferred_element_type=jnp.float32)
        + bm1_ref[...]
    )
    o_ref[...] = jnp.dot(s, wm2_ref[...], preferred_element_type=jnp.float32) + bm2_ref[...]


def _head(h, Wm1, bm1, Wm2, bm2):
    out = pl.pallas_call(
        _head_kernel,
        out_shape=jax.ShapeDtypeStruct((1, 1), jnp.float32),
    )(h, Wm1, bm1.reshape(1, -1), Wm2, bm2.reshape(1, 1))
    return out[0]


def _lane_shuffle(v, idx):
    dnums = lax.GatherDimensionNumbers(
        offset_dims=(), collapsed_slice_dims=(0,), start_index_map=(0,))
    return lax.gather(v, idx[:, None], dnums, slice_sizes=(1,),
                      mode=lax.GatherScatterMode.PROMISE_IN_BOUNDS)


def _sc_logits(xl, xr, src_p, dst_p, att):
    mesh = plsc.VectorSubcoreMesh(core_axis_name="c", subcore_axis_name="s")

    @functools.partial(
        pl.kernel,
        out_type=[
            jax.ShapeDtypeStruct((EP,), jnp.float32),
            jax.ShapeDtypeStruct((NW, L), jnp.float32),
        ],
        mesh=mesh,
        scratch_types=[
            pltpu.VMEM((W_E,), jnp.int32),
            pltpu.VMEM((W_E,), jnp.int32),
            pltpu.VMEM((W_E, D), jnp.float32),
            pltpu.VMEM((W_E, D), jnp.float32),
            pltpu.VMEM((W_E,), jnp.float32),
            pltpu.VMEM((L,), jnp.float32),
            pltpu.VMEM((D,), jnp.float32),
            pltpu.SemaphoreType.DMA,
            pltpu.SemaphoreType.DMA,
        ],
    )
    def k(xl_h, xr_h, src_h, dst_h, att_h, lg_h, mx_h,
          src_v, dst_v, xlr, xrr, lg_v, mx_v, att_v, s1, s2):
        cid = lax.axis_index("c")
        sid = lax.axis_index("s")
        wid = sid * NC + cid
        base = wid * CHUNK
        pltpu.sync_copy(att_h, att_v)
        iota = lax.iota(jnp.int32, L)
        att_c = [att_v[pl.ds(c * L, L)] for c in range(D // L)]

        def window(w, running_max):
            off = base + w * W_E
            pltpu.sync_copy(src_h.at[pl.ds(off, W_E)], src_v)
            pltpu.sync_copy(dst_h.at[pl.ds(off, W_E)], dst_v)
            cl = pltpu.async_copy(xl_h.at[src_v], xlr, s1)
            cr = pltpu.async_copy(xr_h.at[dst_v], xrr, s2)
            cl.wait()
            cr.wait()

            def gbody(g, running_max):
                e0 = g * L
                lvec = jnp.zeros((L,), jnp.float32)
                for i in range(L):
                    acc = jnp.zeros((L,), jnp.float32)
                    for c in range(D // L):
                        vl = xlr[e0 + i, pl.ds(c * L, L)]
                        vr = xrr[e0 + i, pl.ds(c * L, L)]
                        v = vl + vr
                        v = jnp.maximum(v, 0.2 * v)
                        acc = acc + v * att_c[c]
                    for sh in (8, 4, 2, 1):
                        acc = acc + _lane_shuffle(acc, iota ^ sh)
                    lvec = jnp.where(iota == i, acc, lvec)
                lg_v[pl.ds(e0, L)] = lvec
                return jnp.maximum(running_max, lvec)

            wmax = lax.fori_loop(0, G, gbody, running_max)
            pltpu.sync_copy(lg_v, lg_h.at[pl.ds(off, W_E)])
            return wmax

        neg = jnp.full((L,), -3e38, jnp.float32)
        running_max = lax.fori_loop(0, N_WIN, window, neg)
        mx_v[...] = running_max
        pltpu.sync_copy(mx_v, mx_h.at[wid])

    return k(xl, xr, src_p, dst_p, att)


NDEN = 10240


def _sc_aggregate(xl, src_p, dst_p, logits_p, maxes):
    mesh = plsc.VectorSubcoreMesh(core_axis_name="c", subcore_axis_name="s")
    ZR = 128

    @functools.partial(
        pl.kernel,
        out_type=[
            jax.ShapeDtypeStruct((NC, NDEN, D), jnp.float32),
            jax.ShapeDtypeStruct((NC, NDEN), jnp.float32),
        ],
        mesh=mesh,
        scratch_types=[
            pltpu.VMEM((W_E,), jnp.int32),
            pltpu.VMEM((W_E,), jnp.int32),
            pltpu.VMEM((W_E, D), jnp.float32),
            pltpu.VMEM((W_E,), jnp.float32),
            pltpu.VMEM((W_E,), jnp.float32),
            pltpu.VMEM((NW, L), jnp.float32),
            pltpu.VMEM((ZR, D), jnp.float32),
            pltpu.VMEM((640,), jnp.float32),
            pltpu.VMEM_SHARED((NDEN, D), jnp.float32),
            pltpu.VMEM_SHARED((NDEN,), jnp.float32),
            pltpu.SemaphoreType.DMA,
        ],
    )
    def k(xl_h, src_h, dst_h, lg_h, mx_h, out_h, den_h,
          src_v, dst_v, rows, lg_v, un_v, mx_v, zb, zd, out_s, den_s, s1):
        cid = lax.axis_index("c")
        sid = lax.axis_index("s")
        wid = sid * NC + cid
        base = wid * CHUNK
        iota = lax.iota(jnp.int32, L)

        pltpu.sync_copy(mx_h, mx_v)
        m = mx_v[0, :]
        for w in range(1, NW):
            m = jnp.maximum(m, mx_v[w, :])
        for sh in (8, 4, 2, 1):
            m = jnp.maximum(m, _lane_shuffle(m, iota ^ sh))

        def zrow(r, _):
            for c in range(D // L):
                zb[r, pl.ds(c * L, L)] = jnp.zeros((L,), jnp.float32)
            return 0
        lax.fori_loop(0, ZR, zrow, 0)
        def zden(r, _):
            zd[pl.ds(r * L, L)] = jnp.zeros((L,), jnp.float32)
            return 0
        lax.fori_loop(0, 640 // L, zden, 0)
        for rep in range(5):
            pltpu.sync_copy(zb, out_s.at[pl.ds(sid * 640 + rep * ZR, ZR)])
        pltpu.sync_copy(zd, den_s.at[pl.ds(sid * 640, 640)])
        plsc.subcore_barrier()

        def window(w, _):
            off = base + w * W_E
            pltpu.sync_copy(src_h.at[pl.ds(off, W_E)], src_v)
            pltpu.sync_copy(dst_h.at[pl.ds(off, W_E)], dst_v)
            pltpu.sync_copy(lg_h.at[pl.ds(off, W_E)], lg_v)
            pltpu.async_copy(xl_h.at[src_v], rows, s1).wait()

            def gbody(g, _):
                e0 = g * L
                lg = lg_v[pl.ds(e0, L)]
                un = jnp.exp(lg - m)
                eg = off + e0 + iota
                un = jnp.where(eg < ETOT, un, 0.0)
                un_v[pl.ds(e0, L)] = un
                for i in range(L):
                    s = un[i]
                    for c in range(D // L):
                        rows[e0 + i, pl.ds(c * L, L)] = (
                            rows[e0 + i, pl.ds(c * L, L)] * s)
                return 0

            lax.fori_loop(0, G, gbody, 0)
            pltpu.sync_copy(un_v, den_s.at[dst_v], add=True)
            pltpu.sync_copy(rows, out_s.at[dst_v], add=True)
            return 0

        lax.fori_loop(0, N_WIN, window, 0)
        plsc.subcore_barrier()

        pltpu.sync_copy(out_s.at[pl.ds(sid * 640, 640)],
                        out_h.at[cid, pl.ds(sid * 640, 640)])
        pltpu.sync_copy(den_s.at[pl.ds(sid * 640, 640)],
                        den_h.at[cid, pl.ds(sid * 640, 640)])

    return k(xl, src_p, dst_p, logits_p, maxes)


def _gatv2_edges(xl, xr, src_p, dst_p, att):
    logits_p, maxes = _sc_logits(xl, xr, src_p, dst_p, att)
    out_p, den_p = _sc_aggregate(xl, src_p, dst_p, logits_p, maxes)
    return out_p, den_p


def _combine_relu_kernel(o0_ref, o1_ref, d0_ref, d1_ref, b_ref, o_ref):
    r = 1.0 / (d0_ref[...] + d1_ref[...] + 1e-16)
    o_ref[...] = jax.nn.relu((o0_ref[...] + o1_ref[...]) * r + b_ref[...])


def _combine(out_p, den_p, bias):
    d0 = den_p[0, :N].reshape(N, 1)
    d1 = den_p[1, :N].reshape(N, 1)
    return pl.pallas_call(
        _combine_relu_kernel,
        out_shape=jax.ShapeDtypeStruct((N, D), jnp.float32),
    )(out_p[0, :N], out_p[1, :N], d0, d1, bias.reshape(1, D))


def _combine_head_kernel(o0_ref, o1_ref, d0_ref, d1_ref, b_ref,
                         wm1_ref, bm1_ref, wm2_ref, bm2_ref, o_ref):
    r = 1.0 / (d0_ref[...] + d1_ref[...] + 1e-16)
    h = (o0_ref[...] + o1_ref[...]) * r + b_ref[...]
    mean_pool = jnp.mean(h, axis=0, keepdims=True)
    max_pool = jnp.max(h, axis=0, keepdims=True)
    emb = jnp.concatenate([mean_pool, max_pool], axis=-1)
    s = jax.nn.relu(
        jnp.dot(emb, wm1_ref[...], preferred_element_type=jnp.float32)
        + bm1_ref[...]
    )
    o_ref[...] = (
        jnp.dot(s, wm2_ref[...], preferred_element_type=jnp.float32)
        + bm2_ref[...]
    )


def _combine_head(out_p, den_p, bias, Wm1, bm1, Wm2, bm2):
    d0 = den_p[0, :N].reshape(N, 1)
    d1 = den_p[1, :N].reshape(N, 1)
    out = pl.pallas_call(
        _combine_head_kernel,
        out_shape=jax.ShapeDtypeStruct((1, 1), jnp.float32),
    )(out_p[0, :N], out_p[1, :N], d0, d1, bias.reshape(1, D),
      Wm1, bm1.reshape(1, -1), Wm2, bm2.reshape(1, 1))
    return out[0]


def kernel(x, edge_index, Wl1, bl1, Wr1, br1, att1, bias1,
           Wl2, bl2, Wr2, br2, att2, bias2, Wm1, bm1, Wm2, bm2):
    n = x.shape[0]
    loop = jnp.arange(n, dtype=edge_index.dtype)
    src = jnp.concatenate([edge_index[0], loop])
    dst = jnp.concatenate([edge_index[1], loop])
    pad = jnp.arange(EP - ETOT, dtype=src.dtype) % n
    src_p = jnp.concatenate([src, pad])
    dst_p = jnp.concatenate([dst, pad])

    xl1 = _dense(x, Wl1, bl1)
    xr1 = _dense(x, Wr1, br1)
    out1, den1 = _gatv2_edges(xl1, xr1, src_p, dst_p, att1)
    h = _combine(out1, den1, bias1)

    xl2 = _dense(h, Wl2, bl2)
    xr2 = _dense(h, Wr2, br2)
    out2, den2 = _gatv2_edges(xl2, xr2, src_p, dst_p, att2)

    return _combine_head(out2, den2, bias2, Wm1, bm1, Wm2, bm2)


# speedup vs baseline: 13.5231x; 1.2069x over previous
import functools

import jax
import jax.numpy as jnp
from jax import lax
from jax.experimental import pallas as pl
from jax.experimental.pallas import tpu as pltpu
from jax.experimental.pallas import tpu_sc as plsc

N = 10000
D = 128
E_RAW = 320000
ETOT = E_RAW + N
NC, NS, L = 2, 16, 16
NW = NC * NS
W_E = 96
CHUNK = 10368
EP = CHUNK * NW
N_WIN = CHUNK // W_E
G = W_E // L


def _matmul_kernel(x_ref, w_ref, b_ref, o_ref):
    o_ref[...] = (
        jnp.dot(x_ref[...], w_ref[...], preferred_element_type=jnp.float32)
        + b_ref[...]
    )


def _dense(x, w, b):
    n, k = x.shape
    m = w.shape[1]
    return pl.pallas_call(
        _matmul_kernel,
        out_shape=jax.ShapeDtypeStruct((n, m), jnp.float32),
    )(x, w, b.reshape(1, m))


def _head_kernel(h_ref, wm1_ref, bm1_ref, wm2_ref, bm2_ref, o_ref):
    h = h_ref[...]
    mean_pool = jnp.mean(h, axis=0, keepdims=True)
    max_pool = jnp.max(h, axis=0, keepdims=True)
    emb = jnp.concatenate([mean_pool, max_pool], axis=-1)
    s = jax.nn.relu(
        jnp.dot(emb, wm1_ref[...], preferred_element_type=jnp.float32)
        + bm1_ref[...]
    )
    o_ref[...] = jnp.dot(s, wm2_ref[...], preferred_element_type=jnp.float32) + bm2_ref[...]


def _head(h, Wm1, bm1, Wm2, bm2):
    out = pl.pallas_call(
        _head_kernel,
        out_shape=jax.ShapeDtypeStruct((1, 1), jnp.float32),
    )(h, Wm1, bm1.reshape(1, -1), Wm2, bm2.reshape(1, 1))
    return out[0]


def _lane_shuffle(v, idx):
    dnums = lax.GatherDimensionNumbers(
        offset_dims=(), collapsed_slice_dims=(0,), start_index_map=(0,))
    return lax.gather(v, idx[:, None], dnums, slice_sizes=(1,),
                      mode=lax.GatherScatterMode.PROMISE_IN_BOUNDS)


NDEN = 10240


def _gatv2_edges(xl, xr, src_p, dst_p, att):
    mesh = plsc.VectorSubcoreMesh(core_axis_name="c", subcore_axis_name="s")
    ZR = 128

    @functools.partial(
        pl.kernel,
        out_type=[
            jax.ShapeDtypeStruct((NC, NDEN, D), jnp.float32),
            jax.ShapeDtypeStruct((NC, NDEN), jnp.float32),
        ],
        mesh=mesh,
        scratch_types=[
            pltpu.VMEM((W_E,), jnp.int32),
            pltpu.VMEM((W_E,), jnp.int32),
            pltpu.VMEM((W_E, D), jnp.float32),
            pltpu.VMEM((W_E, D), jnp.float32),
            pltpu.VMEM((W_E,), jnp.float32),
            pltpu.VMEM((D,), jnp.float32),
            pltpu.VMEM((ZR, D), jnp.float32),
            pltpu.VMEM((640,), jnp.float32),
            pltpu.VMEM_SHARED((NDEN, D), jnp.float32),
            pltpu.VMEM_SHARED((NDEN,), jnp.float32),
            pltpu.SemaphoreType.DMA,
            pltpu.SemaphoreType.DMA,
        ],
    )
    def k(xl_h, xr_h, src_h, dst_h, att_h, out_h, den_h,
          src_v, dst_v, xlr, xrr, un_v, att_v, zb, zd, out_s, den_s, s1, s2):
        cid = lax.axis_index("c")
        sid = lax.axis_index("s")
        wid = sid * NC + cid
        base = wid * CHUNK
        iota = lax.iota(jnp.int32, L)
        pltpu.sync_copy(att_h, att_v)
        att_c = [att_v[pl.ds(c * L, L)] for c in range(D // L)]

        def zrow(r, _):
            for c in range(D // L):
                zb[r, pl.ds(c * L, L)] = jnp.zeros((L,), jnp.float32)
            return 0
        lax.fori_loop(0, ZR, zrow, 0)
        def zden(r, _):
            zd[pl.ds(r * L, L)] = jnp.zeros((L,), jnp.float32)
            return 0
        lax.fori_loop(0, 640 // L, zden, 0)
        for rep in range(5):
            pltpu.sync_copy(zb, out_s.at[pl.ds(sid * 640 + rep * ZR, ZR)])
        pltpu.sync_copy(zd, den_s.at[pl.ds(sid * 640, 640)])
        plsc.subcore_barrier()

        def window(w, _):
            off = base + w * W_E
            pltpu.sync_copy(src_h.at[pl.ds(off, W_E)], src_v)
            pltpu.sync_copy(dst_h.at[pl.ds(off, W_E)], dst_v)
            cl = pltpu.async_copy(xl_h.at[src_v], xlr, s1)
            cr = pltpu.async_copy(xr_h.at[dst_v], xrr, s2)
            cl.wait()
            cr.wait()

            def gbody(g, _):
                e0 = g * L
                unvec = jnp.zeros((L,), jnp.float32)
                for i in range(L):
                    lc = [xlr[e0 + i, pl.ds(c * L, L)] for c in range(D // L)]
                    acc = jnp.zeros((L,), jnp.float32)
                    for c in range(D // L):
                        v = lc[c] + xrr[e0 + i, pl.ds(c * L, L)]
                        v = jnp.maximum(v, 0.2 * v)
                        acc = acc + v * att_c[c]
                    for sh in (8, 4, 2, 1):
                        acc = acc + _lane_shuffle(acc, iota ^ sh)
                    un = jnp.exp(acc)
                    valid = (off + e0 + i) < ETOT
                    un = jnp.where(valid, un, 0.0)
                    for c in range(D // L):
                        xlr[e0 + i, pl.ds(c * L, L)] = lc[c] * un
                    unvec = jnp.where(iota == i, un, unvec)
                un_v[pl.ds(e0, L)] = unvec
                return 0

            lax.fori_loop(0, G, gbody, 0)
            pltpu.sync_copy(un_v, den_s.at[dst_v], add=True)
            pltpu.sync_copy(xlr, out_s.at[dst_v], add=True)
            return 0

        lax.fori_loop(0, N_WIN, window, 0)
        plsc.subcore_barrier()

        pltpu.sync_copy(out_s.at[pl.ds(sid * 640, 640)],
                        out_h.at[cid, pl.ds(sid * 640, 640)])
        pltpu.sync_copy(den_s.at[pl.ds(sid * 640, 640)],
                        den_h.at[cid, pl.ds(sid * 640, 640)])

    return k(xl, xr, src_p, dst_p, att)


def _combine_relu_kernel(o0_ref, o1_ref, d0_ref, d1_ref, b_ref, o_ref):
    r = 1.0 / (d0_ref[...] + d1_ref[...] + 1e-16)
    o_ref[...] = jax.nn.relu((o0_ref[...] + o1_ref[...]) * r + b_ref[...])


def _combine(out_p, den_p, bias):
    d0 = den_p[0, :N].reshape(N, 1)
    d1 = den_p[1, :N].reshape(N, 1)
    return pl.pallas_call(
        _combine_relu_kernel,
        out_shape=jax.ShapeDtypeStruct((N, D), jnp.float32),
    )(out_p[0, :N], out_p[1, :N], d0, d1, bias.reshape(1, D))


def _combine_head_kernel(o0_ref, o1_ref, d0_ref, d1_ref, b_ref,
                         wm1_ref, bm1_ref, wm2_ref, bm2_ref, o_ref):
    r = 1.0 / (d0_ref[...] + d1_ref[...] + 1e-16)
    h = (o0_ref[...] + o1_ref[...]) * r + b_ref[...]
    mean_pool = jnp.mean(h, axis=0, keepdims=True)
    max_pool = jnp.max(h, axis=0, keepdims=True)
    emb = jnp.concatenate([mean_pool, max_pool], axis=-1)
    s = jax.nn.relu(
        jnp.dot(emb, wm1_ref[...], preferred_element_type=jnp.float32)
        + bm1_ref[...]
    )
    o_ref[...] = (
        jnp.dot(s, wm2_ref[...], preferred_element_type=jnp.float32)
        + bm2_ref[...]
    )


def _combine_head(out_p, den_p, bias, Wm1, bm1, Wm2, bm2):
    d0 = den_p[0, :N].reshape(N, 1)
    d1 = den_p[1, :N].reshape(N, 1)
    out = pl.pallas_call(
        _combine_head_kernel,
        out_shape=jax.ShapeDtypeStruct((1, 1), jnp.float32),
    )(out_p[0, :N], out_p[1, :N], d0, d1, bias.reshape(1, D),
      Wm1, bm1.reshape(1, -1), Wm2, bm2.reshape(1, 1))
    return out[0]


def kernel(x, edge_index, Wl1, bl1, Wr1, br1, att1, bias1,
           Wl2, bl2, Wr2, br2, att2, bias2, Wm1, bm1, Wm2, bm2):
    n = x.shape[0]
    loop = jnp.arange(n, dtype=edge_index.dtype)
    src = jnp.concatenate([edge_index[0], loop])
    dst = jnp.concatenate([edge_index[1], loop])
    pad = jnp.arange(EP - ETOT, dtype=src.dtype) % n
    src_p = jnp.concatenate([src, pad])
    dst_p = jnp.concatenate([dst, pad])

    xl1 = _dense(x, Wl1, bl1)
    xr1 = _dense(x, Wr1, br1)
    out1, den1 = _gatv2_edges(xl1, xr1, src_p, dst_p, att1)
    h = _combine(out1, den1, bias1)

    xl2 = _dense(h, Wl2, bl2)
    xr2 = _dense(h, Wr2, br2)
    out2, den2 = _gatv2_edges(xl2, xr2, src_p, dst_p, att2)

    return _combine_head(out2, den2, bias2, Wm1, bm1, Wm2, bm2)


# speedup vs baseline: 14.4264x; 1.0668x over previous
import functools

import jax
import jax.numpy as jnp
from jax import lax
from jax.experimental import pallas as pl
from jax.experimental.pallas import tpu as pltpu
from jax.experimental.pallas import tpu_sc as plsc

N = 10000
D = 128
E_RAW = 320000
ETOT = E_RAW + N
NC, NS, L = 2, 16, 16
NW = NC * NS
W_E = 48
CHUNK = 10368
NBUF = 2
EP = CHUNK * NW
N_WIN = CHUNK // W_E
G = W_E // L


def _matmul_kernel(x_ref, w_ref, b_ref, o_ref):
    o_ref[...] = (
        jnp.dot(x_ref[...], w_ref[...], preferred_element_type=jnp.float32)
        + b_ref[...]
    )


def _dense(x, w, b):
    n, k = x.shape
    m = w.shape[1]
    return pl.pallas_call(
        _matmul_kernel,
        out_shape=jax.ShapeDtypeStruct((n, m), jnp.float32),
    )(x, w, b.reshape(1, m))


def _head_kernel(h_ref, wm1_ref, bm1_ref, wm2_ref, bm2_ref, o_ref):
    h = h_ref[...]
    mean_pool = jnp.mean(h, axis=0, keepdims=True)
    max_pool = jnp.max(h, axis=0, keepdims=True)
    emb = jnp.concatenate([mean_pool, max_pool], axis=-1)
    s = jax.nn.relu(
        jnp.dot(emb, wm1_ref[...], preferred_element_type=jnp.float32)
        + bm1_ref[...]
    )
    o_ref[...] = jnp.dot(s, wm2_ref[...], preferred_element_type=jnp.float32) + bm2_ref[...]


def _head(h, Wm1, bm1, Wm2, bm2):
    out = pl.pallas_call(
        _head_kernel,
        out_shape=jax.ShapeDtypeStruct((1, 1), jnp.float32),
    )(h, Wm1, bm1.reshape(1, -1), Wm2, bm2.reshape(1, 1))
    return out[0]


def _lane_shuffle(v, idx):
    dnums = lax.GatherDimensionNumbers(
        offset_dims=(), collapsed_slice_dims=(0,), start_index_map=(0,))
    return lax.gather(v, idx[:, None], dnums, slice_sizes=(1,),
                      mode=lax.GatherScatterMode.PROMISE_IN_BOUNDS)


NDEN = 10240


def _gatv2_edges(xl, xr, src_p, dst_p, att):
    mesh = plsc.VectorSubcoreMesh(core_axis_name="c", subcore_axis_name="s")
    ZR = 128

    @functools.partial(
        pl.kernel,
        out_type=[
            jax.ShapeDtypeStruct((NC, NDEN, D), jnp.float32),
            jax.ShapeDtypeStruct((NC, NDEN), jnp.float32),
        ],
        mesh=mesh,
        scratch_types=(
            [pltpu.VMEM((W_E,), jnp.int32) for _ in range(NBUF)]
            + [pltpu.VMEM((W_E,), jnp.int32) for _ in range(NBUF)]
            + [pltpu.VMEM((W_E, D), jnp.float32) for _ in range(NBUF)]
            + [pltpu.VMEM((W_E, D), jnp.float32) for _ in range(NBUF)]
            + [pltpu.VMEM((W_E,), jnp.float32) for _ in range(NBUF)]
            + [
                pltpu.VMEM((D,), jnp.float32),
                pltpu.VMEM((640,), jnp.float32),
                pltpu.VMEM_SHARED((NDEN, D), jnp.float32),
                pltpu.VMEM_SHARED((NDEN,), jnp.float32),
            ]
            + [pltpu.SemaphoreType.DMA for _ in range(NBUF)]
            + [pltpu.SemaphoreType.DMA]
        ),
    )
    def k(xl_h, xr_h, src_h, dst_h, att_h, out_h, den_h, *refs):
        srcv = refs[0:NBUF]
        dstv = refs[NBUF:2 * NBUF]
        xlr = refs[2 * NBUF:3 * NBUF]
        xrr = refs[3 * NBUF:4 * NBUF]
        unv = refs[4 * NBUF:5 * NBUF]
        att_v, zd, out_s, den_s = refs[5 * NBUF:5 * NBUF + 4]
        sg = refs[5 * NBUF + 4:5 * NBUF + 4 + NBUF]
        ss = refs[5 * NBUF + 4 + NBUF]
        cid = lax.axis_index("c")
        sid = lax.axis_index("s")
        wid = sid * NC + cid
        base = wid * CHUNK
        iota = lax.iota(jnp.int32, L)
        pltpu.sync_copy(att_h, att_v)
        att_c = [att_v[pl.ds(c * L, L)] for c in range(D // L)]

        def zrow(r, _):
            for c in range(D // L):
                xlr[0][r, pl.ds(c * L, L)] = jnp.zeros((L,), jnp.float32)
            return 0
        lax.fori_loop(0, W_E, zrow, 0)
        def zden(r, _):
            zd[pl.ds(r * L, L)] = jnp.zeros((L,), jnp.float32)
            return 0
        lax.fori_loop(0, 640 // L, zden, 0)
        for rep in range(640 // W_E):
            pltpu.sync_copy(xlr[0], out_s.at[pl.ds(sid * 640 + rep * W_E, W_E)])
        if 640 % W_E:
            pltpu.sync_copy(
                xlr[0].at[pl.ds(0, 640 % W_E)],
                out_s.at[pl.ds(sid * 640 + (640 // W_E) * W_E, 640 % W_E)])
        pltpu.sync_copy(zd, den_s.at[pl.ds(sid * 640, 640)])
        plsc.subcore_barrier()

        def issue(w, b):
            off = base + w * W_E
            pltpu.sync_copy(src_h.at[pl.ds(off, W_E)], srcv[b])
            pltpu.sync_copy(dst_h.at[pl.ds(off, W_E)], dstv[b])
            pltpu.async_copy(xl_h.at[srcv[b]], xlr[b], sg[b])
            pltpu.async_copy(xr_h.at[dstv[b]], xrr[b], sg[b])

        def drain(b):
            pltpu.make_async_copy(xl_h.at[srcv[b]], xlr[b], sg[b]).wait()
            pltpu.make_async_copy(xr_h.at[dstv[b]], xrr[b], sg[b]).wait()

        def compute(w, b):
            off = base + w * W_E

            def gbody(g, _):
                e0 = g * L
                unvec = jnp.zeros((L,), jnp.float32)
                for i in range(L):
                    lc = [xlr[b][e0 + i, pl.ds(c * L, L)]
                          for c in range(D // L)]
                    acc = jnp.zeros((L,), jnp.float32)
                    for c in range(D // L):
                        v = lc[c] + xrr[b][e0 + i, pl.ds(c * L, L)]
                        v = jnp.maximum(v, 0.2 * v)
                        acc = acc + v * att_c[c]
                    for sh in (8, 4, 2, 1):
                        acc = acc + _lane_shuffle(acc, iota ^ sh)
                    un = jnp.exp(acc)
                    valid = (off + e0 + i) < ETOT
                    un = jnp.where(valid, un, 0.0)
                    for c in range(D // L):
                        xlr[b][e0 + i, pl.ds(c * L, L)] = lc[c] * un
                    unvec = jnp.where(iota == i, un, unvec)
                unv[b][pl.ds(e0, L)] = unvec
                return 0

            lax.fori_loop(0, G, gbody, 0)
            c1 = pltpu.async_copy(unv[b], den_s.at[dstv[b]], ss, add=True)
            c2 = pltpu.async_copy(xlr[b], out_s.at[dstv[b]], ss, add=True)
            c1.wait()
            c2.wait()

        def ring(q, _):
            w0 = q * NBUF
            for b in range(NBUF):
                issue(w0 + b, b)
            for b in range(NBUF):
                drain(b)
                compute(w0 + b, b)
            return 0

        lax.fori_loop(0, N_WIN // NBUF, ring, 0)
        plsc.subcore_barrier()

        pltpu.sync_copy(out_s.at[pl.ds(sid * 640, 640)],
                        out_h.at[cid, pl.ds(sid * 640, 640)])
        pltpu.sync_copy(den_s.at[pl.ds(sid * 640, 640)],
                        den_h.at[cid, pl.ds(sid * 640, 640)])

    return k(xl, xr, src_p, dst_p, att)


def _combine_relu_kernel(o0_ref, o1_ref, d0_ref, d1_ref, b_ref, o_ref):
    r = 1.0 / (d0_ref[...] + d1_ref[...] + 1e-16)
    o_ref[...] = jax.nn.relu((o0_ref[...] + o1_ref[...]) * r + b_ref[...])


def _combine(out_p, den_p, bias):
    d0 = den_p[0, :N].reshape(N, 1)
    d1 = den_p[1, :N].reshape(N, 1)
    return pl.pallas_call(
        _combine_relu_kernel,
        out_shape=jax.ShapeDtypeStruct((N, D), jnp.float32),
    )(out_p[0, :N], out_p[1, :N], d0, d1, bias.reshape(1, D))


def _combine_head_kernel(o0_ref, o1_ref, d0_ref, d1_ref, b_ref,
                         wm1_ref, bm1_ref, wm2_ref, bm2_ref, o_ref):
    r = 1.0 / (d0_ref[...] + d1_ref[...] + 1e-16)
    h = (o0_ref[...] + o1_ref[...]) * r + b_ref[...]
    mean_pool = jnp.mean(h, axis=0, keepdims=True)
    max_pool = jnp.max(h, axis=0, keepdims=True)
    emb = jnp.concatenate([mean_pool, max_pool], axis=-1)
    s = jax.nn.relu(
        jnp.dot(emb, wm1_ref[...], preferred_element_type=jnp.float32)
        + bm1_ref[...]
    )
    o_ref[...] = (
        jnp.dot(s, wm2_ref[...], preferred_element_type=jnp.float32)
        + bm2_ref[...]
    )


def _combine_head(out_p, den_p, bias, Wm1, bm1, Wm2, bm2):
    d0 = den_p[0, :N].reshape(N, 1)
    d1 = den_p[1, :N].reshape(N, 1)
    out = pl.pallas_call(
        _combine_head_kernel,
        out_shape=jax.ShapeDtypeStruct((1, 1), jnp.float32),
    )(out_p[0, :N], out_p[1, :N], d0, d1, bias.reshape(1, D),
      Wm1, bm1.reshape(1, -1), Wm2, bm2.reshape(1, 1))
    return out[0]


def kernel(x, edge_index, Wl1, bl1, Wr1, br1, att1, bias1,
           Wl2, bl2, Wr2, br2, att2, bias2, Wm1, bm1, Wm2, bm2):
    n = x.shape[0]
    loop = jnp.arange(n, dtype=edge_index.dtype)
    src = jnp.concatenate([edge_index[0], loop])
    dst = jnp.concatenate([edge_index[1], loop])
    pad = jnp.arange(EP - ETOT, dtype=src.dtype) % n
    src_p = jnp.concatenate([src, pad])
    dst_p = jnp.concatenate([dst, pad])

    xl1 = _dense(x, Wl1, bl1)
    xr1 = _dense(x, Wr1, br1)
    out1, den1 = _gatv2_edges(xl1, xr1, src_p, dst_p, att1)
    h = _combine(out1, den1, bias1)

    xl2 = _dense(h, Wl2, bl2)
    xr2 = _dense(h, Wr2, br2)
    out2, den2 = _gatv2_edges(xl2, xr2, src_p, dst_p, att2)

    return _combine_head(out2, den2, bias2, Wm1, bm1, Wm2, bm2)


# speedup vs baseline: 15.7294x; 1.0903x over previous
import functools

import jax
import jax.numpy as jnp
from jax import lax
from jax.experimental import pallas as pl
from jax.experimental.pallas import tpu as pltpu
from jax.experimental.pallas import tpu_sc as plsc

N = 10000
D = 128
E_RAW = 320000
ETOT = E_RAW + N
NC, NS, L = 2, 16, 16
NW = NC * NS
W_E = 64
CHUNK = 10368
NBUF = 2
EP = CHUNK * NW
N_WIN = CHUNK // W_E
G = W_E // L


def _matmul_kernel(x_ref, w_ref, b_ref, o_ref):
    o_ref[...] = (
        jnp.dot(x_ref[...], w_ref[...], preferred_element_type=jnp.float32)
        + b_ref[...]
    )


def _dense(x, w, b):
    n, k = x.shape
    m = w.shape[1]
    return pl.pallas_call(
        _matmul_kernel,
        out_shape=jax.ShapeDtypeStruct((n, m), jnp.float32),
    )(x, w, b.reshape(1, m))


def _head_kernel(h_ref, wm1_ref, bm1_ref, wm2_ref, bm2_ref, o_ref):
    h = h_ref[...]
    mean_pool = jnp.mean(h, axis=0, keepdims=True)
    max_pool = jnp.max(h, axis=0, keepdims=True)
    emb = jnp.concatenate([mean_pool, max_pool], axis=-1)
    s = jax.nn.relu(
        jnp.dot(emb, wm1_ref[...], preferred_element_type=jnp.float32)
        + bm1_ref[...]
    )
    o_ref[...] = jnp.dot(s, wm2_ref[...], preferred_element_type=jnp.float32) + bm2_ref[...]


def _head(h, Wm1, bm1, Wm2, bm2):
    out = pl.pallas_call(
        _head_kernel,
        out_shape=jax.ShapeDtypeStruct((1, 1), jnp.float32),
    )(h, Wm1, bm1.reshape(1, -1), Wm2, bm2.reshape(1, 1))
    return out[0]


def _lane_shuffle(v, idx):
    dnums = lax.GatherDimensionNumbers(
        offset_dims=(), collapsed_slice_dims=(0,), start_index_map=(0,))
    return lax.gather(v, idx[:, None], dnums, slice_sizes=(1,),
                      mode=lax.GatherScatterMode.PROMISE_IN_BOUNDS)


NDEN = 10240


def _gatv2_edges(xl, xr, src_p, dst_p, att):
    mesh = plsc.VectorSubcoreMesh(core_axis_name="c", subcore_axis_name="s")
    ZR = 128

    @functools.partial(
        pl.kernel,
        out_type=[
            jax.ShapeDtypeStruct((NC, NDEN, D), jnp.float32),
            jax.ShapeDtypeStruct((NC, NDEN), jnp.float32),
        ],
        mesh=mesh,
        scratch_types=(
            [pltpu.VMEM((W_E,), jnp.int32) for _ in range(NBUF)]
            + [pltpu.VMEM((W_E,), jnp.int32) for _ in range(NBUF)]
            + [pltpu.VMEM((W_E, D), jnp.float32) for _ in range(NBUF)]
            + [pltpu.VMEM((W_E, D), jnp.float32) for _ in range(NBUF)]
            + [pltpu.VMEM((W_E,), jnp.float32) for _ in range(NBUF)]
            + [
                pltpu.VMEM((D,), jnp.float32),
                pltpu.VMEM((640,), jnp.float32),
                pltpu.VMEM_SHARED((NDEN, D), jnp.float32),
                pltpu.VMEM_SHARED((NDEN,), jnp.float32),
            ]
            + [pltpu.SemaphoreType.DMA for _ in range(NBUF)]
            + [pltpu.SemaphoreType.DMA]
        ),
    )
    def k(xl_h, xr_h, src_h, dst_h, att_h, out_h, den_h, *refs):
        srcv = refs[0:NBUF]
        dstv = refs[NBUF:2 * NBUF]
        xlr = refs[2 * NBUF:3 * NBUF]
        xrr = refs[3 * NBUF:4 * NBUF]
        unv = refs[4 * NBUF:5 * NBUF]
        att_v, zd, out_s, den_s = refs[5 * NBUF:5 * NBUF + 4]
        sg = refs[5 * NBUF + 4:5 * NBUF + 4 + NBUF]
        ss = refs[5 * NBUF + 4 + NBUF]
        cid = lax.axis_index("c")
        sid = lax.axis_index("s")
        wid = sid * NC + cid
        base = wid * CHUNK
        iota = lax.iota(jnp.int32, L)
        pltpu.sync_copy(att_h, att_v)
        att_c = [att_v[pl.ds(c * L, L)] for c in range(D // L)]

        def zrow(r, _):
            for c in range(D // L):
                xlr[0][r, pl.ds(c * L, L)] = jnp.zeros((L,), jnp.float32)
            return 0
        lax.fori_loop(0, W_E, zrow, 0)
        def zden(r, _):
            zd[pl.ds(r * L, L)] = jnp.zeros((L,), jnp.float32)
            return 0
        lax.fori_loop(0, 640 // L, zden, 0)
        for rep in range(640 // W_E):
            pltpu.sync_copy(xlr[0], out_s.at[pl.ds(sid * 640 + rep * W_E, W_E)])
        if 640 % W_E:
            pltpu.sync_copy(
                xlr[0].at[pl.ds(0, 640 % W_E)],
                out_s.at[pl.ds(sid * 640 + (640 // W_E) * W_E, 640 % W_E)])
        pltpu.sync_copy(zd, den_s.at[pl.ds(sid * 640, 640)])
        plsc.subcore_barrier()

        def issue(w, b):
            off = base + w * W_E
            pltpu.sync_copy(src_h.at[pl.ds(off, W_E)], srcv[b])
            pltpu.sync_copy(dst_h.at[pl.ds(off, W_E)], dstv[b])
            pltpu.async_copy(xl_h.at[srcv[b]], xlr[b], sg[b])
            pltpu.async_copy(xr_h.at[dstv[b]], xrr[b], sg[b])

        def drain(b):
            pltpu.make_async_copy(xl_h.at[srcv[b]], xlr[b], sg[b]).wait()
            pltpu.make_async_copy(xr_h.at[dstv[b]], xrr[b], sg[b]).wait()

        def compute(w, b):
            off = base + w * W_E

            def gbody(g, _):
                e0 = g * L
                unvec = jnp.zeros((L,), jnp.float32)
                for i in range(L):
                    lc = [xlr[b][e0 + i, pl.ds(c * L, L)]
                          for c in range(D // L)]
                    acc = jnp.zeros((L,), jnp.float32)
                    for c in range(D // L):
                        v = lc[c] + xrr[b][e0 + i, pl.ds(c * L, L)]
                        v = jnp.maximum(v, 0.2 * v)
                        acc = acc + v * att_c[c]
                    for sh in (8, 4, 2, 1):
                        acc = acc + _lane_shuffle(acc, iota ^ sh)
                    un = jnp.exp(acc)
                    valid = (off + e0 + i) < ETOT
                    un = jnp.where(valid, un, 0.0)
                    for c in range(D // L):
                        xlr[b][e0 + i, pl.ds(c * L, L)] = lc[c] * un
                    unvec = jnp.where(iota == i, un, unvec)
                unv[b][pl.ds(e0, L)] = unvec
                return 0

            lax.fori_loop(0, G, gbody, 0)
            c1 = pltpu.async_copy(unv[b], den_s.at[dstv[b]], ss, add=True)
            c2 = pltpu.async_copy(xlr[b], out_s.at[dstv[b]], ss, add=True)
            c1.wait()
            c2.wait()

        def ring(q, _):
            w0 = q * NBUF
            for b in range(NBUF):
                issue(w0 + b, b)
            for b in range(NBUF):
                drain(b)
                compute(w0 + b, b)
            return 0

        lax.fori_loop(0, N_WIN // NBUF, ring, 0)
        plsc.subcore_barrier()

        pltpu.sync_copy(out_s.at[pl.ds(sid * 640, 640)],
                        out_h.at[cid, pl.ds(sid * 640, 640)])
        pltpu.sync_copy(den_s.at[pl.ds(sid * 640, 640)],
                        den_h.at[cid, pl.ds(sid * 640, 640)])

    return k(xl, xr, src_p, dst_p, att)


def _combine_relu_kernel(o0_ref, o1_ref, d0_ref, d1_ref, b_ref, o_ref):
    r = 1.0 / (d0_ref[...] + d1_ref[...] + 1e-16)
    o_ref[...] = jax.nn.relu((o0_ref[...] + o1_ref[...]) * r + b_ref[...])


def _combine(out_p, den_p, bias):
    d0 = den_p[0, :N].reshape(N, 1)
    d1 = den_p[1, :N].reshape(N, 1)
    return pl.pallas_call(
        _combine_relu_kernel,
        out_shape=jax.ShapeDtypeStruct((N, D), jnp.float32),
    )(out_p[0, :N], out_p[1, :N], d0, d1, bias.reshape(1, D))


def _combine_head_kernel(o0_ref, o1_ref, d0_ref, d1_ref, b_ref,
                         wm1_ref, bm1_ref, wm2_ref, bm2_ref, o_ref):
    r = 1.0 / (d0_ref[...] + d1_ref[...] + 1e-16)
    h = (o0_ref[...] + o1_ref[...]) * r + b_ref[...]
    mean_pool = jnp.mean(h, axis=0, keepdims=True)
    max_pool = jnp.max(h, axis=0, keepdims=True)
    emb = jnp.concatenate([mean_pool, max_pool], axis=-1)
    s = jax.nn.relu(
        jnp.dot(emb, wm1_ref[...], preferred_element_type=jnp.float32)
        + bm1_ref[...]
    )
    o_ref[...] = (
        jnp.dot(s, wm2_ref[...], preferred_element_type=jnp.float32)
        + bm2_ref[...]
    )


def _combine_head(out_p, den_p, bias, Wm1, bm1, Wm2, bm2):
    d0 = den_p[0, :N].reshape(N, 1)
    d1 = den_p[1, :N].reshape(N, 1)
    out = pl.pallas_call(
        _combine_head_kernel,
        out_shape=jax.ShapeDtypeStruct((1, 1), jnp.float32),
    )(out_p[0, :N], out_p[1, :N], d0, d1, bias.reshape(1, D),
      Wm1, bm1.reshape(1, -1), Wm2, bm2.reshape(1, 1))
    return out[0]


def kernel(x, edge_index, Wl1, bl1, Wr1, br1, att1, bias1,
           Wl2, bl2, Wr2, br2, att2, bias2, Wm1, bm1, Wm2, bm2):
    n = x.shape[0]
    loop = jnp.arange(n, dtype=edge_index.dtype)
    src = jnp.concatenate([edge_index[0], loop])
    dst = jnp.concatenate([edge_index[1], loop])
    pad = jnp.arange(EP - ETOT, dtype=src.dtype) % n
    src_p = jnp.concatenate([src, pad])
    dst_p = jnp.concatenate([dst, pad])

    xl1 = _dense(x, Wl1, bl1)
    xr1 = _dense(x, Wr1, br1)
    out1, den1 = _gatv2_edges(xl1, xr1, src_p, dst_p, att1)
    h = _combine(out1, den1, bias1)

    xl2 = _dense(h, Wl2, bl2)
    xr2 = _dense(h, Wr2, br2)
    out2, den2 = _gatv2_edges(xl2, xr2, src_p, dst_p, att2)

    return _combine_head(out2, den2, bias2, Wm1, bm1, Wm2, bm2)
